# Initial kernel scaffold; baseline (speedup 1.0000x reference)
import jax
import jax.numpy as jnp
from jax.experimental import pallas as pl


def kernel(t_startXs, t_startYs, img_prev, img_next):
    raise NotImplementedError("write your pallas kernel here")


# speedup vs baseline: 16170.6670x; 16170.6670x over previous
import functools

import jax
import jax.numpy as jnp
import numpy as np
from jax import lax
from jax.experimental import pallas as pl
from jax.experimental.pallas import tpu as pltpu
from jax.experimental.pallas import tpu_sc as plsc

WIN = 25
NELEM = WIN * WIN
NCHUNK = 40
WBUF = NCHUNK * 16
LEVELS = 15
H = W = 512
BAND = 29
HALF = 12.0

_x = np.arange(5, dtype=np.float32) - 2
_g = np.exp(-(_x ** 2) / np.float32(2.0 * 0.2 ** 2)).astype(np.float32)
_g = _g / _g.sum()
_K2 = np.outer(_g, _g).astype(np.float32)


def _preprocess_body(prev_ref, next_ref, img1_ref, img2_ref, ix_ref, iy_ref):
    p = prev_ref[...]
    gray_p = 0.299 * p[0] + 0.587 * p[1] + 0.114 * p[2]
    n = next_ref[...]
    gray_n = 0.299 * n[0] + 0.587 * n[1] + 0.114 * n[2]
    img1_ref[...] = gray_p * 255.0
    img2_ref[...] = gray_n * 255.0

    g = gray_p
    g = jnp.concatenate(
        [g[2:3], g[1:2], g, g[H - 2:H - 1], g[H - 3:H - 2]], axis=0)
    g = jnp.concatenate(
        [g[:, 2:3], g[:, 1:2], g, g[:, W - 2:W - 1], g[:, W - 3:W - 2]],
        axis=1)
    acc = jnp.zeros((H, W), jnp.float32)
    for i in range(5):
        for j in range(5):
            acc = acc + _K2[i, j] * g[i:i + H, j:j + W]
    I = acc * 255.0

    iy = jnp.concatenate(
        [I[1:2] - I[0:1], (I[2:] - I[:-2]) / 2.0, I[H - 1:H] - I[H - 2:H - 1]],
        axis=0)
    ix = jnp.concatenate(
        [I[:, 1:2] - I[:, 0:1], (I[:, 2:] - I[:, :-2]) / 2.0,
         I[:, W - 1:W] - I[:, W - 2:W - 1]], axis=1)
    iy_ref[...] = iy
    ix_ref[...] = ix


def _bf16r(v):
    b = plsc.bitcast(v, jnp.int32)
    lsb = lax.shift_right_logical(b, 16) & 1
    r = (b + 0x7FFF + lsb) & jnp.int32(-65536)
    return plsc.bitcast(r, jnp.float32)


def _bf16s(x):
    return _bf16r(jnp.full((16,), x, jnp.float32))[0]


def _sdiv(a, b):
    return (jnp.full((16,), a, jnp.float32) / jnp.full((16,), b, jnp.float32))[0]


def _floor_i32(x):
    xi = x.astype(jnp.int32)
    return xi - jnp.where(xi.astype(jnp.float32) > x, 1, 0).astype(jnp.int32)


def _band_base(yp):
    yc = jnp.clip(yp, -1.0e6, 1.0e6)
    yi = yc.astype(jnp.int32)
    yi = yi - jnp.where(yi.astype(jnp.float32) > yc, 1, 0).astype(jnp.int32)
    return jnp.clip(yi - 14, 0, H - BAND)


def _chunk_sample(band_ref, y_lo, xp, yp, k):
    e = lax.iota(jnp.int32, 16) + k * 16
    jx = lax.shift_right_logical(e * 5243, 17)
    jy = e - jx * WIN
    xq = (jx.astype(jnp.float32) + xp) - HALF
    yq = (jy.astype(jnp.float32) + yp) - HALF
    xq = jnp.clip(xq, -1.0e6, 1.0e6)
    yq = jnp.clip(yq, -1.0e6, 1.0e6)
    xf = _floor_i32(xq)
    yf = _floor_i32(yq)
    xc = xf + jnp.where(xq > xf.astype(jnp.float32), 1, 0).astype(jnp.int32)
    yc = yf + jnp.where(yq > yf.astype(jnp.float32), 1, 0).astype(jnp.int32)
    xfc = jnp.clip(xf, 0, W - 1)
    xcc = jnp.clip(xc, 0, W - 1)
    yfc = jnp.clip(yf, 0, H - 1)
    ycc = jnp.clip(yc, 0, H - 1)
    lw = xq - xfc.astype(jnp.float32)
    hw = 1.0 - lw
    lh = yq - yfc.astype(jnp.float32)
    hh = 1.0 - lh
    rf = jnp.clip(yfc - y_lo, 0, BAND - 1) * W
    rc = jnp.clip(ycc - y_lo, 0, BAND - 1) * W
    v1 = plsc.load_gather(band_ref, [rf + xfc])
    v2 = plsc.load_gather(band_ref, [rf + xcc])
    v3 = plsc.load_gather(band_ref, [rc + xfc])
    v4 = plsc.load_gather(band_ref, [rc + xcc])
    return v1 * (hh * hw) + (hh * lw) * v2 + (lh * hw) * v3 + (lh * lw) * v4


def _build_window(img_hbm, band_ref, dest_ref, base, xp, yp):
    y_lo = _band_base(yp)
    pltpu.sync_copy(img_hbm.at[pl.ds(y_lo * W, BAND * W)], band_ref)

    def body(k, _):
        val = _chunk_sample(band_ref, y_lo, xp, yp, k)
        e = lax.iota(jnp.int32, 16) + k * 16
        val = jnp.where(e < NELEM, val, 0.0)
        dest_ref[pl.ds(base + k * 16, 16)] = val
        return 0

    lax.fori_loop(0, NCHUNK, body, 0)


def _sc_body(xs, ys, img1, img2, ixm, iym, outx, outy,
             posx, posy, band, w1b, wxb, wyb, iox, ioy):
    pltpu.sync_copy(xs.at[pl.ds(0, 16)], posx)
    pltpu.sync_copy(ys.at[pl.ds(0, 16)], posy)
    posxv = posx[...]
    posyv = posy[...]
    p0x = posxv[0]
    p1x = posxv[1]
    p0y = posyv[0]
    p1y = posyv[1]

    inv_rows = []
    for p, (xp, yp) in enumerate(((p0x, p0y), (p1x, p1y))):
        base = p * WBUF
        _build_window(img1, band, w1b, base, xp, yp)
        _build_window(ixm, band, wxb, base, xp, yp)
        _build_window(iym, band, wyb, base, xp, yp)

        def acc_body(k, carry, base=base):
            axx, axy, ayy = carry
            wx = _bf16r(wxb[pl.ds(base + k * 16, 16)])
            wy = _bf16r(wyb[pl.ds(base + k * 16, 16)])
            return (axx + wx * wx, axy + wx * wy, ayy + wy * wy)

        z = jnp.zeros((16,), jnp.float32)
        axx, axy, ayy = lax.fori_loop(0, NCHUNK, acc_body, (z, z, z))
        a11 = jnp.sum(axx)
        a12 = jnp.sum(axy)
        a22 = jnp.sum(ayy)
        det = a11 * a22 - a12 * a12
        inv_rows.append((_bf16s(_sdiv(a22, det)), _bf16s(_sdiv(-a12, det))))

    def step(_, carry):
        x0, y0, x1, y1, dxt, dyt = carry
        sols = []
        for p, (xp, yp) in enumerate(((x0, y0), (x1, y1))):
            base = p * WBUF
            y_lo = _band_base(yp)
            pltpu.sync_copy(img2.at[pl.ds(y_lo * W, BAND * W)], band)

            def dot_body(k, carry2, base=base, y_lo=y_lo, xp=xp, yp=yp):
                ax, ay = carry2
                val = _chunk_sample(band, y_lo, xp, yp, k)
                diff = _bf16r(val - w1b[pl.ds(base + k * 16, 16)])
                wx = _bf16r(wxb[pl.ds(base + k * 16, 16)])
                wy = _bf16r(wyb[pl.ds(base + k * 16, 16)])
                return (ax + wx * diff, ay + wy * diff)

            z = jnp.zeros((16,), jnp.float32)
            ax, ay = lax.fori_loop(0, NCHUNK, dot_body, (z, z))
            bx = _bf16s(-jnp.sum(ax))
            by = _bf16s(-jnp.sum(ay))
            i11, i12 = inv_rows[p]
            sols.append(i11 * bx + i12 * by)
        dx, dy = sols[0], sols[1]
        return (x0 + dx, y0 + dy, x1 + dx, y1 + dy, dxt + dx, dyt + dy)

    zero = jnp.float32(0.0)
    init = (p0x, p0y, p1x, p1y, zero, zero)
    _, _, _, _, dxt, dyt = lax.fori_loop(0, LEVELS, step, init)

    nc = 2
    wid = lax.axis_index("c") * 16 + lax.axis_index("s")
    chunk = 4096 // (nc * 16)
    bs = wid * chunk
    pltpu.sync_copy(xs.at[pl.ds(bs, chunk)], iox)
    pltpu.sync_copy(ys.at[pl.ds(bs, chunk)], ioy)
    dxv = jnp.full((16,), dxt, jnp.float32)
    dyv = jnp.full((16,), dyt, jnp.float32)

    def add_body(i, _):
        iox[pl.ds(i * 16, 16)] = iox[pl.ds(i * 16, 16)] + dxv
        ioy[pl.ds(i * 16, 16)] = ioy[pl.ds(i * 16, 16)] + dyv
        return 0

    lax.fori_loop(0, chunk // 16, add_body, 0)
    pltpu.sync_copy(iox, outx.at[pl.ds(bs, chunk)])
    pltpu.sync_copy(ioy, outy.at[pl.ds(bs, chunk)])


def kernel(t_startXs, t_startYs, img_prev, img_next):
    imgf = jax.ShapeDtypeStruct((H, W), jnp.float32)
    img1, img2, ix, iy = pl.pallas_call(
        _preprocess_body,
        out_shape=[imgf, imgf, imgf, imgf],
    )(img_prev, img_next)

    outf = jax.ShapeDtypeStruct((4096,), jnp.float32)
    mesh = plsc.VectorSubcoreMesh(core_axis_name="c", subcore_axis_name="s")
    sc = functools.partial(
        pl.kernel,
        mesh=mesh,
        compiler_params=pltpu.CompilerParams(needs_layout_passes=False),
        out_type=[outf, outf],
        scratch_types=[
            pltpu.VMEM((16,), jnp.float32),
            pltpu.VMEM((16,), jnp.float32),
            pltpu.VMEM((BAND * W,), jnp.float32),
            pltpu.VMEM((2 * WBUF,), jnp.float32),
            pltpu.VMEM((2 * WBUF,), jnp.float32),
            pltpu.VMEM((2 * WBUF,), jnp.float32),
            pltpu.VMEM((128,), jnp.float32),
            pltpu.VMEM((128,), jnp.float32),
        ],
    )(_sc_body)
    newXs, newYs = sc(t_startXs, t_startYs, img1.reshape(-1), img2.reshape(-1),
                      ix.reshape(-1), iy.reshape(-1))
    return (newXs, newYs)


# speedup vs baseline: 16271.0553x; 1.0062x over previous
import functools

import jax
import jax.numpy as jnp
import numpy as np
from jax import lax
from jax.experimental import pallas as pl
from jax.experimental.pallas import tpu as pltpu
from jax.experimental.pallas import tpu_sc as plsc

WIN = 25
NELEM = WIN * WIN
NCHUNK = 40
WBUF = NCHUNK * 16
LEVELS = 15
H = W = 512
BAND = 29
HALF = 12.0

_x = np.arange(5, dtype=np.float32) - 2
_g = np.exp(-(_x ** 2) / np.float32(2.0 * 0.2 ** 2)).astype(np.float32)
_g = _g / _g.sum()
_K2 = np.outer(_g, _g).astype(np.float32)


def _preprocess_body(prev_ref, next_ref, img1_ref, img2_ref, ix_ref, iy_ref):
    p = prev_ref[...]
    gray_p = 0.299 * p[0] + 0.587 * p[1] + 0.114 * p[2]
    n = next_ref[...]
    gray_n = 0.299 * n[0] + 0.587 * n[1] + 0.114 * n[2]
    img1_ref[...] = gray_p * 255.0
    img2_ref[...] = gray_n * 255.0

    g = gray_p
    g = jnp.concatenate(
        [g[2:3], g[1:2], g, g[H - 2:H - 1], g[H - 3:H - 2]], axis=0)
    g = jnp.concatenate(
        [g[:, 2:3], g[:, 1:2], g, g[:, W - 2:W - 1], g[:, W - 3:W - 2]],
        axis=1)
    acc = jnp.zeros((H, W), jnp.float32)
    for i in range(5):
        for j in range(5):
            acc = acc + _K2[i, j] * g[i:i + H, j:j + W]
    I = acc * 255.0

    iy = jnp.concatenate(
        [I[1:2] - I[0:1], (I[2:] - I[:-2]) / 2.0, I[H - 1:H] - I[H - 2:H - 1]],
        axis=0)
    ix = jnp.concatenate(
        [I[:, 1:2] - I[:, 0:1], (I[:, 2:] - I[:, :-2]) / 2.0,
         I[:, W - 1:W] - I[:, W - 2:W - 1]], axis=1)
    iy_ref[...] = iy
    ix_ref[...] = ix


def _bf16r(v):
    b = plsc.bitcast(v, jnp.int32)
    lsb = lax.shift_right_logical(b, 16) & 1
    r = (b + 0x7FFF + lsb) & jnp.int32(-65536)
    return plsc.bitcast(r, jnp.float32)


def _sdiv(a, b):
    return (jnp.full((16,), a, jnp.float32) / jnp.full((16,), b, jnp.float32))[0]


def _floor_i32(x):
    xi = x.astype(jnp.int32)
    return xi - jnp.where(xi.astype(jnp.float32) > x, 1, 0).astype(jnp.int32)


def _band_base(yp):
    yc = jnp.clip(yp, -1.0e6, 1.0e6)
    yi = yc.astype(jnp.int32)
    yi = yi - jnp.where(yi.astype(jnp.float32) > yc, 1, 0).astype(jnp.int32)
    return jnp.clip(yi - 14, 0, H - BAND)


def _chunk_sample(band_ref, y_lo, xp, yp, k):
    e = lax.iota(jnp.int32, 16) + k * 16
    jx = lax.shift_right_logical(e * 5243, 17)
    jy = e - jx * WIN
    xq = (jx.astype(jnp.float32) + xp) - HALF
    yq = (jy.astype(jnp.float32) + yp) - HALF
    xq = jnp.clip(xq, -1.0e6, 1.0e6)
    yq = jnp.clip(yq, -1.0e6, 1.0e6)
    xf = _floor_i32(xq)
    yf = _floor_i32(yq)
    xc = xf + jnp.where(xq > xf.astype(jnp.float32), 1, 0).astype(jnp.int32)
    yc = yf + jnp.where(yq > yf.astype(jnp.float32), 1, 0).astype(jnp.int32)
    xfc = jnp.clip(xf, 0, W - 1)
    xcc = jnp.clip(xc, 0, W - 1)
    yfc = jnp.clip(yf, 0, H - 1)
    ycc = jnp.clip(yc, 0, H - 1)
    lw = xq - xfc.astype(jnp.float32)
    hw = 1.0 - lw
    lh = yq - yfc.astype(jnp.float32)
    hh = 1.0 - lh
    rf = jnp.clip(yfc - y_lo, 0, BAND - 1) * W
    rc = jnp.clip(ycc - y_lo, 0, BAND - 1) * W
    v1 = plsc.load_gather(band_ref, [rf + xfc])
    v2 = plsc.load_gather(band_ref, [rf + xcc])
    v3 = plsc.load_gather(band_ref, [rc + xfc])
    v4 = plsc.load_gather(band_ref, [rc + xcc])
    return v1 * (hh * hw) + (hh * lw) * v2 + (lh * hw) * v3 + (lh * lw) * v4


def _build_window(img_hbm, band_ref, dest_ref, base, xp, yp):
    y_lo = _band_base(yp)
    pltpu.sync_copy(img_hbm.at[pl.ds(y_lo * W, BAND * W)], band_ref)

    def body(k, _):
        val = _chunk_sample(band_ref, y_lo, xp, yp, k)
        e = lax.iota(jnp.int32, 16) + k * 16
        val = jnp.where(e < NELEM, val, 0.0)
        dest_ref[pl.ds(base + k * 16, 16)] = val
        return 0

    lax.fori_loop(0, NCHUNK, body, 0)


def _sc_body(xs, ys, img1, img2, ixm, iym, outx, outy,
             posx, posy, band, w1b, wxb, wyb, iox, ioy):
    pltpu.sync_copy(xs.at[pl.ds(0, 16)], posx)
    pltpu.sync_copy(ys.at[pl.ds(0, 16)], posy)
    posxv = posx[...]
    posyv = posy[...]
    p0x = posxv[0]
    p1x = posxv[1]
    p0y = posyv[0]
    p1y = posyv[1]

    inv_rows = []
    for p, (xp, yp) in enumerate(((p0x, p0y), (p1x, p1y))):
        base = p * WBUF
        _build_window(img1, band, w1b, base, xp, yp)
        _build_window(ixm, band, wxb, base, xp, yp)
        _build_window(iym, band, wyb, base, xp, yp)

        def acc_body(k, carry, base=base):
            axx, axy, ayy = carry
            wx = _bf16r(wxb[pl.ds(base + k * 16, 16)])
            wy = _bf16r(wyb[pl.ds(base + k * 16, 16)])
            return (axx + wx * wx, axy + wx * wy, ayy + wy * wy)

        z = jnp.zeros((16,), jnp.float32)
        axx, axy, ayy = lax.fori_loop(0, NCHUNK, acc_body, (z, z, z))
        a11 = jnp.sum(axx)
        a12 = jnp.sum(axy)
        a22 = jnp.sum(ayy)
        det = a11 * a22 - a12 * a12
        inv_rows.append((_sdiv(a22, det), _sdiv(-a12, det)))

    def step(_, carry):
        x0, y0, x1, y1, dxt, dyt = carry
        sols = []
        for p, (xp, yp) in enumerate(((x0, y0), (x1, y1))):
            base = p * WBUF
            y_lo = _band_base(yp)
            pltpu.sync_copy(img2.at[pl.ds(y_lo * W, BAND * W)], band)

            def dot_body(k, carry2, base=base, y_lo=y_lo, xp=xp, yp=yp):
                ax, ay = carry2
                val = _chunk_sample(band, y_lo, xp, yp, k)
                diff = _bf16r(val - w1b[pl.ds(base + k * 16, 16)])
                wx = _bf16r(wxb[pl.ds(base + k * 16, 16)])
                wy = _bf16r(wyb[pl.ds(base + k * 16, 16)])
                return (ax + wx * diff, ay + wy * diff)

            z = jnp.zeros((16,), jnp.float32)
            ax, ay = lax.fori_loop(0, NCHUNK, dot_body, (z, z))
            bx = -jnp.sum(ax)
            by = -jnp.sum(ay)
            i11, i12 = inv_rows[p]
            sols.append(i11 * bx + i12 * by)
        dx, dy = sols[0], sols[1]
        return (x0 + dx, y0 + dy, x1 + dx, y1 + dy, dxt + dx, dyt + dy)

    zero = jnp.float32(0.0)
    init = (p0x, p0y, p1x, p1y, zero, zero)
    _, _, _, _, dxt, dyt = lax.fori_loop(0, LEVELS, step, init)

    nc = 2
    wid = lax.axis_index("c") * 16 + lax.axis_index("s")
    chunk = 4096 // (nc * 16)
    bs = wid * chunk
    pltpu.sync_copy(xs.at[pl.ds(bs, chunk)], iox)
    pltpu.sync_copy(ys.at[pl.ds(bs, chunk)], ioy)
    dxv = jnp.full((16,), dxt, jnp.float32)
    dyv = jnp.full((16,), dyt, jnp.float32)

    def add_body(i, _):
        iox[pl.ds(i * 16, 16)] = iox[pl.ds(i * 16, 16)] + dxv
        ioy[pl.ds(i * 16, 16)] = ioy[pl.ds(i * 16, 16)] + dyv
        return 0

    lax.fori_loop(0, chunk // 16, add_body, 0)
    pltpu.sync_copy(iox, outx.at[pl.ds(bs, chunk)])
    pltpu.sync_copy(ioy, outy.at[pl.ds(bs, chunk)])


def kernel(t_startXs, t_startYs, img_prev, img_next):
    imgf = jax.ShapeDtypeStruct((H, W), jnp.float32)
    img1, img2, ix, iy = pl.pallas_call(
        _preprocess_body,
        out_shape=[imgf, imgf, imgf, imgf],
    )(img_prev, img_next)

    outf = jax.ShapeDtypeStruct((4096,), jnp.float32)
    mesh = plsc.VectorSubcoreMesh(core_axis_name="c", subcore_axis_name="s")
    sc = functools.partial(
        pl.kernel,
        mesh=mesh,
        compiler_params=pltpu.CompilerParams(needs_layout_passes=False),
        out_type=[outf, outf],
        scratch_types=[
            pltpu.VMEM((16,), jnp.float32),
            pltpu.VMEM((16,), jnp.float32),
            pltpu.VMEM((BAND * W,), jnp.float32),
            pltpu.VMEM((2 * WBUF,), jnp.float32),
            pltpu.VMEM((2 * WBUF,), jnp.float32),
            pltpu.VMEM((2 * WBUF,), jnp.float32),
            pltpu.VMEM((128,), jnp.float32),
            pltpu.VMEM((128,), jnp.float32),
        ],
    )(_sc_body)
    newXs, newYs = sc(t_startXs, t_startYs, img1.reshape(-1), img2.reshape(-1),
                      ix.reshape(-1), iy.reshape(-1))
    return (newXs, newYs)


# speedup vs baseline: 17614.7228x; 1.0826x over previous
import functools

import jax
import jax.numpy as jnp
import numpy as np
from jax import lax
from jax.experimental import pallas as pl
from jax.experimental.pallas import tpu as pltpu
from jax.experimental.pallas import tpu_sc as plsc

WIN = 25
NELEM = WIN * WIN
NCHUNK = 40
WBUF = NCHUNK * 16
LEVELS = 15
H = W = 512
BAND = 29
HALF = 12.0

_x = np.arange(5, dtype=np.float32) - 2
_g = np.exp(-(_x ** 2) / np.float32(2.0 * 0.2 ** 2)).astype(np.float32)
_g = _g / _g.sum()
_K2 = np.outer(_g, _g).astype(np.float32)


def _preprocess_body(prev_ref, next_ref, img1_ref, img2_ref, ix_ref, iy_ref):
    p = prev_ref[...]
    gray_p = 0.299 * p[0] + 0.587 * p[1] + 0.114 * p[2]
    n = next_ref[...]
    gray_n = 0.299 * n[0] + 0.587 * n[1] + 0.114 * n[2]
    img1_ref[...] = gray_p * 255.0
    img2_ref[...] = gray_n * 255.0

    g = gray_p
    g = jnp.concatenate(
        [g[2:3], g[1:2], g, g[H - 2:H - 1], g[H - 3:H - 2]], axis=0)
    g = jnp.concatenate(
        [g[:, 2:3], g[:, 1:2], g, g[:, W - 2:W - 1], g[:, W - 3:W - 2]],
        axis=1)
    acc = jnp.zeros((H, W), jnp.float32)
    for i in range(5):
        for j in range(5):
            acc = acc + _K2[i, j] * g[i:i + H, j:j + W]
    I = acc * 255.0

    iy = jnp.concatenate(
        [I[1:2] - I[0:1], (I[2:] - I[:-2]) / 2.0, I[H - 1:H] - I[H - 2:H - 1]],
        axis=0)
    ix = jnp.concatenate(
        [I[:, 1:2] - I[:, 0:1], (I[:, 2:] - I[:, :-2]) / 2.0,
         I[:, W - 1:W] - I[:, W - 2:W - 1]], axis=1)
    iy_ref[...] = iy
    ix_ref[...] = ix


def _bf16r(v):
    b = plsc.bitcast(v, jnp.int32)
    lsb = lax.shift_right_logical(b, 16) & 1
    r = (b + 0x7FFF + lsb) & jnp.int32(-65536)
    return plsc.bitcast(r, jnp.float32)


def _sdiv(a, b):
    return (jnp.full((16,), a, jnp.float32) / jnp.full((16,), b, jnp.float32))[0]


def _floor_i32(x):
    xi = x.astype(jnp.int32)
    return xi - jnp.where(xi.astype(jnp.float32) > x, 1, 0).astype(jnp.int32)


def _band_base(yp):
    yc = jnp.clip(yp, -1.0e6, 1.0e6)
    yi = yc.astype(jnp.int32)
    yi = yi - jnp.where(yi.astype(jnp.float32) > yc, 1, 0).astype(jnp.int32)
    return jnp.clip(yi - 14, 0, H - BAND)


def _chunk_sample(band_ref, y_lo, xp, yp, k):
    e = lax.iota(jnp.int32, 16) + k * 16
    jx = lax.shift_right_logical(e * 5243, 17)
    jy = e - jx * WIN
    xq = (jx.astype(jnp.float32) + xp) - HALF
    yq = (jy.astype(jnp.float32) + yp) - HALF
    xq = jnp.clip(xq, -1.0e6, 1.0e6)
    yq = jnp.clip(yq, -1.0e6, 1.0e6)
    xf = _floor_i32(xq)
    yf = _floor_i32(yq)
    xc = xf + jnp.where(xq > xf.astype(jnp.float32), 1, 0).astype(jnp.int32)
    yc = yf + jnp.where(yq > yf.astype(jnp.float32), 1, 0).astype(jnp.int32)
    xfc = jnp.clip(xf, 0, W - 1)
    xcc = jnp.clip(xc, 0, W - 1)
    yfc = jnp.clip(yf, 0, H - 1)
    ycc = jnp.clip(yc, 0, H - 1)
    lw = xq - xfc.astype(jnp.float32)
    hw = 1.0 - lw
    lh = yq - yfc.astype(jnp.float32)
    hh = 1.0 - lh
    rf = jnp.clip(yfc - y_lo, 0, BAND - 1) * W
    rc = jnp.clip(ycc - y_lo, 0, BAND - 1) * W
    v1 = plsc.load_gather(band_ref, [rf + xfc])
    v2 = plsc.load_gather(band_ref, [rf + xcc])
    v3 = plsc.load_gather(band_ref, [rc + xfc])
    v4 = plsc.load_gather(band_ref, [rc + xcc])
    return v1 * (hh * hw) + (hh * lw) * v2 + (lh * hw) * v3 + (lh * lw) * v4


def _sample_window(band_ref, y_lo, dest_ref, base, xp, yp):
    def body(k, _):
        val = _chunk_sample(band_ref, y_lo, xp, yp, k)
        e = lax.iota(jnp.int32, 16) + k * 16
        val = jnp.where(e < NELEM, val, 0.0)
        dest_ref[pl.ds(base + k * 16, 16)] = val
        return 0

    lax.fori_loop(0, NCHUNK, body, 0)


def _sc_body(xs, ys, img1, img2, ixm, iym, outx, outy,
             posx, posy, band0, band1, w1b, wxb, wyb, iox, ioy, sem0, sem1):
    pltpu.sync_copy(xs.at[pl.ds(0, 16)], posx)
    pltpu.sync_copy(ys.at[pl.ds(0, 16)], posy)
    posxv = posx[...]
    posyv = posy[...]
    p0x = posxv[0]
    p1x = posxv[1]
    p0y = posyv[0]
    p1y = posyv[1]
    y_lo0 = _band_base(p0y)
    y_lo1 = _band_base(p1y)

    for img, dest in ((img1, w1b), (ixm, wxb), (iym, wyb)):
        cp0 = pltpu.async_copy(img.at[pl.ds(y_lo0 * W, BAND * W)], band0, sem0)
        cp1 = pltpu.async_copy(img.at[pl.ds(y_lo1 * W, BAND * W)], band1, sem1)
        cp0.wait()
        _sample_window(band0, y_lo0, dest, 0, p0x, p0y)
        cp1.wait()
        _sample_window(band1, y_lo1, dest, WBUF, p1x, p1y)

    inv_rows = []
    for p in (0, 1):
        base = p * WBUF

        def acc_body(k, carry, base=base):
            axx, axy, ayy = carry
            wx = _bf16r(wxb[pl.ds(base + k * 16, 16)])
            wy = _bf16r(wyb[pl.ds(base + k * 16, 16)])
            return (axx + wx * wx, axy + wx * wy, ayy + wy * wy)

        z = jnp.zeros((16,), jnp.float32)
        axx, axy, ayy = lax.fori_loop(0, NCHUNK, acc_body, (z, z, z))
        a11 = jnp.sum(axx)
        a12 = jnp.sum(axy)
        a22 = jnp.sum(ayy)
        det = a11 * a22 - a12 * a12
        inv_rows.append((_sdiv(a22, det), _sdiv(-a12, det)))

    def step(_, carry):
        x0, y0, x1, y1, dxt, dyt = carry
        ylo0 = _band_base(y0)
        ylo1 = _band_base(y1)
        c0 = pltpu.async_copy(img2.at[pl.ds(ylo0 * W, BAND * W)], band0, sem0)
        c1 = pltpu.async_copy(img2.at[pl.ds(ylo1 * W, BAND * W)], band1, sem1)
        sols = []
        for p, (band, y_lo, cp, xp, yp) in enumerate((
                (band0, ylo0, c0, x0, y0), (band1, ylo1, c1, x1, y1))):
            base = p * WBUF
            cp.wait()

            def dot_body(k, carry2, band=band, base=base, y_lo=y_lo, xp=xp, yp=yp):
                ax, ay = carry2
                val = _chunk_sample(band, y_lo, xp, yp, k)
                diff = _bf16r(val - w1b[pl.ds(base + k * 16, 16)])
                wx = _bf16r(wxb[pl.ds(base + k * 16, 16)])
                wy = _bf16r(wyb[pl.ds(base + k * 16, 16)])
                return (ax + wx * diff, ay + wy * diff)

            z = jnp.zeros((16,), jnp.float32)
            ax, ay = lax.fori_loop(0, NCHUNK, dot_body, (z, z))
            bx = -jnp.sum(ax)
            by = -jnp.sum(ay)
            i11, i12 = inv_rows[p]
            sols.append(i11 * bx + i12 * by)
        dx, dy = sols[0], sols[1]
        return (x0 + dx, y0 + dy, x1 + dx, y1 + dy, dxt + dx, dyt + dy)

    zero = jnp.float32(0.0)
    init = (p0x, p0y, p1x, p1y, zero, zero)
    _, _, _, _, dxt, dyt = lax.fori_loop(0, LEVELS, step, init)

    nc = 2
    wid = lax.axis_index("c") * 16 + lax.axis_index("s")
    chunk = 4096 // (nc * 16)
    bs = wid * chunk
    pltpu.sync_copy(xs.at[pl.ds(bs, chunk)], iox)
    pltpu.sync_copy(ys.at[pl.ds(bs, chunk)], ioy)
    dxv = jnp.full((16,), dxt, jnp.float32)
    dyv = jnp.full((16,), dyt, jnp.float32)

    def add_body(i, _):
        iox[pl.ds(i * 16, 16)] = iox[pl.ds(i * 16, 16)] + dxv
        ioy[pl.ds(i * 16, 16)] = ioy[pl.ds(i * 16, 16)] + dyv
        return 0

    lax.fori_loop(0, chunk // 16, add_body, 0)
    pltpu.sync_copy(iox, outx.at[pl.ds(bs, chunk)])
    pltpu.sync_copy(ioy, outy.at[pl.ds(bs, chunk)])


def kernel(t_startXs, t_startYs, img_prev, img_next):
    imgf = jax.ShapeDtypeStruct((H, W), jnp.float32)
    img1, img2, ix, iy = pl.pallas_call(
        _preprocess_body,
        out_shape=[imgf, imgf, imgf, imgf],
    )(img_prev, img_next)

    outf = jax.ShapeDtypeStruct((4096,), jnp.float32)
    mesh = plsc.VectorSubcoreMesh(core_axis_name="c", subcore_axis_name="s")
    sc = functools.partial(
        pl.kernel,
        mesh=mesh,
        compiler_params=pltpu.CompilerParams(needs_layout_passes=False),
        out_type=[outf, outf],
        scratch_types=[
            pltpu.VMEM((16,), jnp.float32),
            pltpu.VMEM((16,), jnp.float32),
            pltpu.VMEM((BAND * W,), jnp.float32),
            pltpu.VMEM((BAND * W,), jnp.float32),
            pltpu.VMEM((2 * WBUF,), jnp.float32),
            pltpu.VMEM((2 * WBUF,), jnp.float32),
            pltpu.VMEM((2 * WBUF,), jnp.float32),
            pltpu.VMEM((128,), jnp.float32),
            pltpu.VMEM((128,), jnp.float32),
            pltpu.SemaphoreType.DMA,
            pltpu.SemaphoreType.DMA,
        ],
    )(_sc_body)
    newXs, newYs = sc(t_startXs, t_startYs, img1.reshape(-1), img2.reshape(-1),
                      ix.reshape(-1), iy.reshape(-1))
    return (newXs, newYs)
